# baseline (speedup 1.0000x reference)
import jax
import jax.numpy as jnp
from jax.experimental import pallas as pl
from jax.experimental.pallas import tpu as pltpu

_K = 49152
_DIM = 256
_B = 4096
_NBLK = _K // _B
_LW = 128
_LR = _B // _LW


def _body(ptr_ref, keys_ref, labels_ref, outq_ref, outl_ref, outp_ref,
          zq_ref, zl_ref, semq, seml):
    ptr = ptr_ref[0]

    pltpu.make_async_copy(keys_ref, outq_ref.at[pl.ds(0, _B), :], semq.at[0]).start()
    pltpu.make_async_copy(labels_ref, outl_ref.at[pl.ds(0, _LR), :], seml.at[0]).start()

    zq_ref[...] = jnp.zeros_like(zq_ref)
    zl_ref[...] = jnp.zeros_like(zl_ref)
    for i in range(1, _NBLK):
        pltpu.make_async_copy(
            zq_ref, outq_ref.at[pl.ds(i * _B, _B), :], semq.at[i]).start()
        pltpu.make_async_copy(
            zl_ref, outl_ref.at[pl.ds(i * _LR, _LR), :], seml.at[i]).start()

    outp_ref[0] = jnp.mod(ptr + _B, _K)

    pltpu.make_async_copy(keys_ref, outq_ref.at[pl.ds(0, _B), :], semq.at[0]).wait()
    pltpu.make_async_copy(labels_ref, outl_ref.at[pl.ds(0, _LR), :], seml.at[0]).wait()
    for i in range(1, _NBLK):
        pltpu.make_async_copy(
            zq_ref, outq_ref.at[pl.ds(i * _B, _B), :], semq.at[i]).wait()
        pltpu.make_async_copy(
            zl_ref, outl_ref.at[pl.ds(i * _LR, _LR), :], seml.at[i]).wait()


def kernel(source_features, source_labels, queue, queue_labels, queue_ptr):
    del queue, queue_labels
    labels2 = source_labels.reshape(_LR, _LW)
    newq, newl, newp = pl.pallas_call(
        _body,
        in_specs=[
            pl.BlockSpec(memory_space=pltpu.SMEM),
            pl.BlockSpec((_B, _DIM), lambda: (0, 0)),
            pl.BlockSpec((_LR, _LW), lambda: (0, 0)),
        ],
        out_specs=[
            pl.BlockSpec(memory_space=pl.ANY),
            pl.BlockSpec(memory_space=pl.ANY),
            pl.BlockSpec(memory_space=pltpu.SMEM),
        ],
        out_shape=[
            jax.ShapeDtypeStruct((_K, _DIM), jnp.float32),
            jax.ShapeDtypeStruct((_K // _LW, _LW), jnp.int32),
            jax.ShapeDtypeStruct((1,), jnp.int32),
        ],
        scratch_shapes=[
            pltpu.VMEM((_B, _DIM), jnp.float32),
            pltpu.VMEM((_LR, _LW), jnp.int32),
            pltpu.SemaphoreType.DMA((_NBLK,)),
            pltpu.SemaphoreType.DMA((_NBLK,)),
        ],
    )(queue_ptr, source_features, labels2)
    return newq, newl.reshape(_K), newp
